# baseline (speedup 1.0000x reference)
import jax
import jax.numpy as jnp
from jax import lax
from jax.experimental import pallas as pl
from jax.experimental.pallas import tpu as pltpu
from jax.experimental.pallas import tpu_sc as plsc

_N = 8192
_D = 2048
_NC = 2
_NS = 16
_NW = _NC * _NS
_ROWS_PER_W = _N // _NW
_R = 4
_CHUNKS = _ROWS_PER_W // _R
_NBUF = 6
_MAIN = _CHUNKS // _NBUF
_TAIL = _CHUNKS - _MAIN * _NBUF
_L = 16
_JG = _D // _L


def _permute_chunk(idx_v, src, dst):
    @plsc.parallel_loop(0, _JG, 1, unroll=8)
    def _(jg):
        col = idx_v[pl.ds(jg * _L, _L)]
        for r in range(_R):
            row_i = jnp.full((_L,), r, dtype=jnp.int32)
            dst[r, pl.ds(jg * _L, _L)] = plsc.load_gather(src, [row_i, col])


def _sc_body(x_hbm, idx_hbm, out_hbm, idx_v,
             in0, in1, in2, in3, in4, in5, out0, out1, out2, out3, out4, out5,
             si0, si1, si2, si3, si4, si5, so0, so1, so2, so3, so4, so5):
    wid = lax.axis_index("s") * _NC + lax.axis_index("c")
    base = wid * _ROWS_PER_W
    pltpu.sync_copy(idx_hbm, idx_v)

    ins = (in0, in1, in2, in3, in4, in5)
    outs = (out0, out1, out2, out3, out4, out5)
    sis = (si0, si1, si2, si3, si4, si5)
    sos = (so0, so1, so2, so3, so4, so5)

    for k in range(_NBUF):
        pltpu.async_copy(x_hbm.at[pl.ds(base + k * _R, _R)], ins[k], sis[k])

    def ring_body(i, carry):
        for k in range(_NBUF):
            row = base + (i * _NBUF + k) * _R
            pltpu.make_async_copy(x_hbm.at[pl.ds(row, _R)], ins[k],
                                  sis[k]).wait()

            @pl.when(i > 0)
            def _(k=k, row=row):
                pltpu.make_async_copy(outs[k],
                                      out_hbm.at[pl.ds(row - _NBUF * _R, _R)],
                                      sos[k]).wait()

            _permute_chunk(idx_v, ins[k], outs[k])
            pltpu.async_copy(outs[k], out_hbm.at[pl.ds(row, _R)], sos[k])

            if k < _TAIL:
                pltpu.async_copy(x_hbm.at[pl.ds(row + _NBUF * _R, _R)],
                                 ins[k], sis[k])
            else:
                @pl.when(i < _MAIN - 1)
                def _(k=k, row=row):
                    pltpu.async_copy(x_hbm.at[pl.ds(row + _NBUF * _R, _R)],
                                     ins[k], sis[k])

        return carry

    lax.fori_loop(0, _MAIN, ring_body, 0)

    for k in range(_TAIL):
        row = base + (_MAIN * _NBUF + k) * _R
        pltpu.make_async_copy(x_hbm.at[pl.ds(row, _R)], ins[k], sis[k]).wait()
        pltpu.make_async_copy(outs[k], out_hbm.at[pl.ds(row - _NBUF * _R, _R)],
                              sos[k]).wait()
        _permute_chunk(idx_v, ins[k], outs[k])
        pltpu.async_copy(outs[k], out_hbm.at[pl.ds(row, _R)], sos[k])

    for k in range(_NBUF):
        c = _MAIN * _NBUF + k if k < _TAIL else (_MAIN - 1) * _NBUF + k
        pltpu.make_async_copy(outs[k], out_hbm.at[pl.ds(base + c * _R, _R)],
                              sos[k]).wait()


@jax.jit
def kernel(inputs, idxs):
    mesh = plsc.VectorSubcoreMesh(
        core_axis_name="c", subcore_axis_name="s",
        num_cores=_NC, num_subcores=_NS,
    )
    f = pl.kernel(
        _sc_body,
        out_type=jax.ShapeDtypeStruct((_N, _D), jnp.float32),
        mesh=mesh,
        scratch_types=[
            pltpu.VMEM((_D,), jnp.int32),
            pltpu.VMEM((_R, _D), jnp.float32),
            pltpu.VMEM((_R, _D), jnp.float32),
            pltpu.VMEM((_R, _D), jnp.float32),
            pltpu.VMEM((_R, _D), jnp.float32),
            pltpu.VMEM((_R, _D), jnp.float32),
            pltpu.VMEM((_R, _D), jnp.float32),
            pltpu.VMEM((_R, _D), jnp.float32),
            pltpu.VMEM((_R, _D), jnp.float32),
            pltpu.VMEM((_R, _D), jnp.float32),
            pltpu.VMEM((_R, _D), jnp.float32),
            pltpu.VMEM((_R, _D), jnp.float32),
            pltpu.VMEM((_R, _D), jnp.float32),
            pltpu.SemaphoreType.DMA,
            pltpu.SemaphoreType.DMA,
            pltpu.SemaphoreType.DMA,
            pltpu.SemaphoreType.DMA,
            pltpu.SemaphoreType.DMA,
            pltpu.SemaphoreType.DMA,
            pltpu.SemaphoreType.DMA,
            pltpu.SemaphoreType.DMA,
            pltpu.SemaphoreType.DMA,
            pltpu.SemaphoreType.DMA,
            pltpu.SemaphoreType.DMA,
            pltpu.SemaphoreType.DMA,
        ],
        compiler_params=pltpu.CompilerParams(needs_layout_passes=False),
    )
    return f(inputs, idxs)


# speedup vs baseline: 1.0128x; 1.0128x over previous
import jax
import jax.numpy as jnp
from jax import lax
from jax.experimental import pallas as pl
from jax.experimental.pallas import tpu as pltpu
from jax.experimental.pallas import tpu_sc as plsc

_N = 8192
_D = 2048
_NC = 2
_NS = 16
_NW = _NC * _NS
_ROWS_PER_W = _N // _NW
_R = 4
_CHUNKS = _ROWS_PER_W // _R
_NBUF = 4
_MAIN = _CHUNKS // _NBUF
_TAIL = _CHUNKS - _MAIN * _NBUF
_L = 16
_JG = _D // _L


def _permute_chunk(idx_v, src, dst):
    @plsc.parallel_loop(0, _JG, 1, unroll=8)
    def _(jg):
        col = idx_v[pl.ds(jg * _L, _L)]
        for r in range(_R):
            row_i = jnp.full((_L,), r, dtype=jnp.int32)
            dst[r, pl.ds(jg * _L, _L)] = plsc.load_gather(src, [row_i, col])


def _sc_body(x_hbm, idx_hbm, out_hbm, idx_v,
             in0, in1, in2, in3, out0, out1, out2, out3,
             si0, si1, si2, si3, so0, so1, so2, so3):
    wid = lax.axis_index("s") * _NC + lax.axis_index("c")
    base = wid * _ROWS_PER_W
    pltpu.sync_copy(idx_hbm, idx_v)

    ins = (in0, in1, in2, in3)
    outs = (out0, out1, out2, out3)
    sis = (si0, si1, si2, si3)
    sos = (so0, so1, so2, so3)

    for k in range(_NBUF):
        pltpu.async_copy(x_hbm.at[pl.ds(base + k * _R, _R)], ins[k], sis[k])

    def ring_body(i, carry):
        for k in range(_NBUF):
            row = base + (i * _NBUF + k) * _R
            pltpu.make_async_copy(x_hbm.at[pl.ds(row, _R)], ins[k],
                                  sis[k]).wait()

            @pl.when(i > 0)
            def _(k=k, row=row):
                pltpu.make_async_copy(outs[k],
                                      out_hbm.at[pl.ds(row - _NBUF * _R, _R)],
                                      sos[k]).wait()

            _permute_chunk(idx_v, ins[k], outs[k])
            pltpu.async_copy(outs[k], out_hbm.at[pl.ds(row, _R)], sos[k])

            if k < _TAIL:
                pltpu.async_copy(x_hbm.at[pl.ds(row + _NBUF * _R, _R)],
                                 ins[k], sis[k])
            else:
                @pl.when(i < _MAIN - 1)
                def _(k=k, row=row):
                    pltpu.async_copy(x_hbm.at[pl.ds(row + _NBUF * _R, _R)],
                                     ins[k], sis[k])

        return carry

    lax.fori_loop(0, _MAIN, ring_body, 0)

    for k in range(_TAIL):
        row = base + (_MAIN * _NBUF + k) * _R
        pltpu.make_async_copy(x_hbm.at[pl.ds(row, _R)], ins[k], sis[k]).wait()
        pltpu.make_async_copy(outs[k], out_hbm.at[pl.ds(row - _NBUF * _R, _R)],
                              sos[k]).wait()
        _permute_chunk(idx_v, ins[k], outs[k])
        pltpu.async_copy(outs[k], out_hbm.at[pl.ds(row, _R)], sos[k])

    for k in range(_NBUF):
        c = _MAIN * _NBUF + k if k < _TAIL else (_MAIN - 1) * _NBUF + k
        pltpu.make_async_copy(outs[k], out_hbm.at[pl.ds(base + c * _R, _R)],
                              sos[k]).wait()


@jax.jit
def kernel(inputs, idxs):
    mesh = plsc.VectorSubcoreMesh(
        core_axis_name="c", subcore_axis_name="s",
        num_cores=_NC, num_subcores=_NS,
    )
    f = pl.kernel(
        _sc_body,
        out_type=jax.ShapeDtypeStruct((_N, _D), jnp.float32),
        mesh=mesh,
        scratch_types=[
            pltpu.VMEM((_D,), jnp.int32),
            pltpu.VMEM((_R, _D), jnp.float32),
            pltpu.VMEM((_R, _D), jnp.float32),
            pltpu.VMEM((_R, _D), jnp.float32),
            pltpu.VMEM((_R, _D), jnp.float32),
            pltpu.VMEM((_R, _D), jnp.float32),
            pltpu.VMEM((_R, _D), jnp.float32),
            pltpu.VMEM((_R, _D), jnp.float32),
            pltpu.VMEM((_R, _D), jnp.float32),
            pltpu.SemaphoreType.DMA,
            pltpu.SemaphoreType.DMA,
            pltpu.SemaphoreType.DMA,
            pltpu.SemaphoreType.DMA,
            pltpu.SemaphoreType.DMA,
            pltpu.SemaphoreType.DMA,
            pltpu.SemaphoreType.DMA,
            pltpu.SemaphoreType.DMA,
        ],
        compiler_params=pltpu.CompilerParams(needs_layout_passes=False),
    )
    return f(inputs, idxs)
